# Initial kernel scaffold; baseline (speedup 1.0000x reference)
import jax
import jax.numpy as jnp
from jax.experimental import pallas as pl


def kernel(x, edge_index, eps1, w1a, b1a, w1b, b1b, eps2, w2a, b2a, w2b, b2b):
    raise NotImplementedError("write your pallas kernel here")


# speedup vs baseline: 3.7703x; 3.7703x over previous
import functools

import jax
import jax.numpy as jnp
from jax import lax
from jax.experimental import pallas as pl
from jax.experimental.pallas import tpu as pltpu
from jax.experimental.pallas import tpu_sc as plsc

N = 10000
F = 128
E = 320000
NC = 2
NS = 16
CHUNK = 128
CH_PER_TILE = 79
EDGES_PER_TILE = CHUNK * CH_PER_TILE
E_PAD = EDGES_PER_TILE * NC * NS
ACC_ROWS = 10112
RPT = ACC_ROWS // NS


def _sc_aggregate(table, src, dst, zeros):
    mesh = plsc.VectorSubcoreMesh(core_axis_name="c", subcore_axis_name="s")

    @functools.partial(
        pl.kernel,
        mesh=mesh,
        out_type=jax.ShapeDtypeStruct((NC, N, F), jnp.float32),
        scratch_types=[
            pltpu.VMEM_SHARED((ACC_ROWS, F), jnp.float32),
            pltpu.VMEM((CHUNK,), jnp.int32),
            pltpu.VMEM((CHUNK,), jnp.int32),
            pltpu.VMEM((CHUNK, F), jnp.float32),
            pltpu.SemaphoreType.DMA,
        ],
    )
    def agg(table_hbm, src_hbm, dst_hbm, zeros_hbm, parts_hbm,
            acc, src_v, dst_v, rows_v, sem):
        cid = lax.axis_index("c")
        sid = lax.axis_index("s")
        wid = cid * NS + sid

        pltpu.sync_copy(zeros_hbm, acc.at[pl.ds(sid * RPT, RPT)])
        plsc.subcore_barrier()

        ebase = wid * EDGES_PER_TILE

        def body(j, carry):
            base = ebase + j * CHUNK
            pltpu.sync_copy(src_hbm.at[pl.ds(base, CHUNK)], src_v)
            pltpu.sync_copy(dst_hbm.at[pl.ds(base, CHUNK)], dst_v)
            pltpu.async_copy(table_hbm.at[src_v], rows_v, sem).wait()
            pltpu.sync_copy(rows_v, acc.at[dst_v], add=True)
            return carry

        lax.fori_loop(0, CH_PER_TILE, body, 0)
        plsc.subcore_barrier()

        row0 = sid * RPT

        @pl.when(sid < NS - 1)
        def _():
            pltpu.sync_copy(acc.at[pl.ds(row0, RPT)],
                            parts_hbm.at[cid, pl.ds(row0, RPT)])

        @pl.when(sid == NS - 1)
        def _():
            last = N - (NS - 1) * RPT
            pltpu.sync_copy(acc.at[pl.ds(row0, last)],
                            parts_hbm.at[cid, pl.ds(row0, last)])

    return agg(table, src, dst, zeros)


BLK = 1000


def _tc_mlp(x, parts, eps, wa, ba, wb, bb, final_relu):
    scale = jnp.reshape(1.0 + eps, (1, 1)).astype(jnp.float32)
    ba2 = ba.reshape(1, F)
    bb2 = bb.reshape(1, F)

    def body(x_ref, p_ref, s_ref, wa_ref, ba_ref, wb_ref, bb_ref, o_ref):
        h = s_ref[0, 0] * x_ref[...] + p_ref[0] + p_ref[1]
        t = jnp.dot(h, wa_ref[...], preferred_element_type=jnp.float32)
        t = jnp.maximum(t + ba_ref[...], 0.0)
        t = jnp.dot(t, wb_ref[...], preferred_element_type=jnp.float32)
        t = t + bb_ref[...]
        if final_relu:
            t = jnp.maximum(t, 0.0)
        o_ref[...] = t

    return pl.pallas_call(
        body,
        grid=(N // BLK,),
        in_specs=[
            pl.BlockSpec((BLK, F), lambda i: (i, 0)),
            pl.BlockSpec((NC, BLK, F), lambda i: (0, i, 0)),
            pl.BlockSpec((1, 1), lambda i: (0, 0)),
            pl.BlockSpec((F, F), lambda i: (0, 0)),
            pl.BlockSpec((1, F), lambda i: (0, 0)),
            pl.BlockSpec((F, F), lambda i: (0, 0)),
            pl.BlockSpec((1, F), lambda i: (0, 0)),
        ],
        out_specs=pl.BlockSpec((BLK, F), lambda i: (i, 0)),
        out_shape=jax.ShapeDtypeStruct((N, F), jnp.float32),
    )(x, parts, scale, wa, ba2, wb, bb2)


def kernel(x, edge_index, eps1, w1a, b1a, w1b, b1b, eps2, w2a, b2a, w2b, b2b):
    src = edge_index[0]
    dst = edge_index[1]
    pad = E_PAD - E
    src_p = jnp.concatenate([src, jnp.zeros((pad,), jnp.int32)])
    dst_p = jnp.concatenate([dst, jnp.full((pad,), N, jnp.int32)])
    zeros = jnp.zeros((RPT, F), jnp.float32)

    parts1 = _sc_aggregate(x, src_p, dst_p, zeros)
    h1 = _tc_mlp(x, parts1, eps1, w1a, b1a, w1b, b1b, final_relu=True)
    parts2 = _sc_aggregate(h1, src_p, dst_p, zeros)
    out = _tc_mlp(h1, parts2, eps2, w2a, b2a, w2b, b2b, final_relu=False)
    return out
